# Initial kernel scaffold; baseline (speedup 1.0000x reference)
import jax
import jax.numpy as jnp
from jax.experimental import pallas as pl


def kernel(intensity):
    raise NotImplementedError("write your pallas kernel here")


# speedup vs baseline: 24.2465x; 24.2465x over previous
import functools

import jax
import jax.numpy as jnp
from jax.experimental import pallas as pl
from jax.experimental.pallas import tpu as pltpu

_STEEPNESS = 10.0
_ROWS_PER_BLOCK = 8
_BISECT_ITERS = 20


def _rw_kernel(x_ref, mask_ref, thr_ref, *, k):
    x = x_ref[...]
    kf = jnp.float32(k)
    lo = jnp.min(x, axis=1, keepdims=True)
    hi = jnp.max(x, axis=1, keepdims=True)

    def body(_, lohi):
        lo, hi = lohi
        t = 0.5 * (lo + hi)
        c = jnp.sum((x >= t).astype(jnp.float32), axis=1, keepdims=True)
        ge = c >= kf
        return jnp.where(ge, t, lo), jnp.where(ge, hi, t)

    lo, hi = jax.lax.fori_loop(0, _BISECT_ITERS, body, (lo, hi))
    t = lo
    gt = x > t
    cgt = jnp.sum(gt.astype(jnp.float32), axis=1, keepdims=True)
    sgt = jnp.sum(jnp.where(gt, x, 0.0), axis=1, keepdims=True)
    thr = 2.0 * (sgt + (kf - cgt) * t) / kf
    thr_ref[...] = thr
    mask_ref[...] = jax.nn.sigmoid(_STEEPNESS * (x - thr))


def kernel(intensity):
    B, H, W = intensity.shape
    N = H * W
    k = max(1, N // 3)
    R = _ROWS_PER_BLOCK
    flat = intensity.reshape(B, N)
    grid = (B // R,)
    mask, thr = pl.pallas_call(
        functools.partial(_rw_kernel, k=k),
        grid=grid,
        in_specs=[pl.BlockSpec((R, N), lambda i: (i, 0))],
        out_specs=[
            pl.BlockSpec((R, N), lambda i: (i, 0)),
            pl.BlockSpec((R, 1), lambda i: (i, 0)),
        ],
        out_shape=[
            jax.ShapeDtypeStruct((B, N), jnp.float32),
            jax.ShapeDtypeStruct((B, 1), jnp.float32),
        ],
        compiler_params=pltpu.CompilerParams(
            dimension_semantics=("parallel",),
        ),
    )(flat)
    mask = mask.reshape(B, H, W)
    thr = thr.reshape(B, 1, 1)
    return (mask, thr, mask)
